# baseline (speedup 1.0000x reference)
import functools

import jax
import jax.numpy as jnp
from jax import lax
from jax.experimental import pallas as pl
from jax.experimental.pallas import tpu as pltpu
from jax.experimental.pallas import tpu_sc as plsc

N_IDS = 1000000
EMBED_DIM = 32
BATCH = 16384
HIST = 50

NC = 2
NS = 16
NW = NC * NS
ROWS_PER_W = BATCH // NW
ROWS_PER_CHUNK = 2
CHUNK = ROWS_PER_CHUNK * HIST
NCHUNKS = ROWS_PER_W // ROWS_PER_CHUNK
HALF = EMBED_DIM // 2


def _sc_embedding_sum(idx4, table2):
  mesh = plsc.VectorSubcoreMesh(core_axis_name="c", subcore_axis_name="s")

  @functools.partial(
      pl.kernel,
      mesh=mesh,
      out_type=jax.ShapeDtypeStruct((BATCH, EMBED_DIM), jnp.float32),
      compiler_params=pltpu.CompilerParams(use_tc_tiling_on_sc=False),
      scratch_types=[
          pltpu.VMEM((NCHUNKS, 2, CHUNK), jnp.int32),
          pltpu.VMEM((2 * CHUNK, HALF), jnp.float32),
          pltpu.VMEM((2 * CHUNK, HALF), jnp.float32),
          pltpu.VMEM((2 * CHUNK, HALF), jnp.float32),
          pltpu.VMEM((2 * CHUNK, HALF), jnp.float32),
          pltpu.VMEM((ROWS_PER_W, EMBED_DIM), jnp.float32),
          pltpu.SemaphoreType.DMA,
          pltpu.SemaphoreType.DMA,
          pltpu.SemaphoreType.DMA,
          pltpu.SemaphoreType.DMA,
      ],
  )
  def k(idx_hbm, table_hbm, out_hbm, idx_v, buf0, buf1, buf2, buf3, out_v,
        sem0, sem1, sem2, sem3):
    bufs = (buf0, buf1, buf2, buf3)
    sems = (sem0, sem1, sem2, sem3)
    nbuf = 4

    wid = lax.axis_index("s") * NC + lax.axis_index("c")

    pltpu.sync_copy(idx_hbm.at[wid], idx_v)

    def start(c, buf, sem):
      pltpu.async_copy(table_hbm.at[idx_v.at[c, 0]], buf.at[pl.ds(0, CHUNK)],
                       sem)
      pltpu.async_copy(table_hbm.at[idx_v.at[c, 1]],
                       buf.at[pl.ds(CHUNK, CHUNK)], sem)

    def wait(buf, sem):
      pltpu.make_async_copy(table_hbm.at[idx_v.at[0, 0]], buf, sem).wait()

    def accumulate(buf, local_row0):
      for g in range(ROWS_PER_CHUNK):
        base = g * HIST
        a0 = buf[base, pl.ds(0, 16)]
        a1 = buf[CHUNK + base, pl.ds(0, 16)]
        for j in range(1, HIST):
          a0 = a0 + buf[base + j, pl.ds(0, 16)]
          a1 = a1 + buf[CHUNK + base + j, pl.ds(0, 16)]
        out_v[local_row0 + g, pl.ds(0, 16)] = a0
        out_v[local_row0 + g, pl.ds(16, 16)] = a1

    for c in range(nbuf - 1):
      start(c, bufs[c], sems[c])

    def body(i, _):
      for k in range(nbuf):
        c = nbuf * i + k
        ahead = c + nbuf - 1

        @pl.when(ahead < NCHUNKS)
        def _():
          start(ahead, bufs[(k + nbuf - 1) % nbuf], sems[(k + nbuf - 1) % nbuf])

        wait(bufs[k], sems[k])
        accumulate(bufs[k], ROWS_PER_CHUNK * c)
      return 0

    lax.fori_loop(0, NCHUNKS // nbuf, body, 0)

    pltpu.sync_copy(out_v, out_hbm.at[pl.ds(wid * ROWS_PER_W, ROWS_PER_W)])

  return k(idx4, table2)


def kernel(inputs, W):
  idx = inputs.astype(jnp.int32).reshape(NW, NCHUNKS, 1, CHUNK)
  idx4 = jnp.concatenate([2 * idx, 2 * idx + 1], axis=2)
  table2 = W.reshape(2 * N_IDS, HALF)
  return _sc_embedding_sum(idx4, table2)
